# Initial kernel scaffold; baseline (speedup 1.0000x reference)
import jax
import jax.numpy as jnp
from jax.experimental import pallas as pl


def kernel(cF, sF, cmasks, smasks, s_c1w, s_c1b, s_c2w, s_c2b, s_c3w, s_c3b, s_fcw, s_fcb, c_c1w, c_c1b, c_c2w, c_c2b, c_c3w, c_c3b, c_fcw, c_fcb, comp_w, comp_b, unzip_w, unzip_b):
    raise NotImplementedError("write your pallas kernel here")


# speedup vs baseline: 1.9588x; 1.9588x over previous
import numpy as np
import jax
import jax.numpy as jnp
from jax.experimental import pallas as pl
from jax.experimental.pallas import tpu as pltpu

H = W = 64
HW = H * W
NM = 9

_INTERPRET = False

_TAPS = []
_VMASK_NP = np.zeros((9, HW), dtype=np.float32)
for _kh in range(3):
    for _kw in range(3):
        _dy, _dx = _kh - 1, _kw - 1
        _delta = _dy * W + _dx
        _hh, _ww = np.meshgrid(np.arange(H), np.arange(W), indexing="ij")
        _valid = ((_hh + _dy >= 0) & (_hh + _dy < H)
                  & (_ww + _dx >= 0) & (_ww + _dx < W))
        _k = _kh * 3 + _kw
        _VMASK_NP[_k] = _valid.reshape(-1).astype(np.float32)
        _TAPS.append(((-_delta) % HW, _k))


def _last_valid_onehot(cond):
    notafter = jnp.ones((1, HW), dtype=jnp.float32)
    rows = [None] * NM
    for i in range(NM - 1, -1, -1):
        ci = cond[i:i + 1, :]
        rows[i] = ci * notafter
        notafter = notafter * (1.0 - ci)
    return jnp.concatenate(rows, axis=0)


def _dot(a, b):
    return jax.lax.dot_general(a, b, (((1,), (0,)), ((), ())),
                               preferred_element_type=jnp.float32)


def _dot_t(a, b):
    return jax.lax.dot_general(a, b, (((1,), (1,)), ((), ())),
                               preferred_element_type=jnp.float32)


def _conv3x3(h, w_ref, b, vm, relu=True):
    acc = None
    for roll_amt, k in _TAPS:
        wk = w_ref[k]
        if roll_amt == 0:
            xs = h
        else:
            xs = jnp.roll(h, roll_amt, axis=1) * vm[k:k + 1, :]
        t = _dot(wk, xs)
        acc = t if acc is None else acc + t
    acc = acc + b
    return jnp.maximum(acc, 0.0) if relu else acc


def _branch_body(x_ref, m_ref, vm_ref, w1_ref, b1_ref, w2_ref, b2_ref,
                 w3_ref, b3_ref, covs_ref, fsm_ref, cnt_ref):
    x = x_ref[...]
    m = m_ref[...]
    vm = vm_ref[...]

    cnt = jnp.sum(m, axis=1, keepdims=True)
    inv = 1.0 / jnp.maximum(cnt, 1.0)
    ok = (cnt >= 10.0).astype(jnp.float32)

    sums = _dot_t(x, m)
    cond = m * ok
    S = _last_valid_onehot(cond)
    fsm = x - _dot(sums, S * inv)
    fsm_ref[...] = fsm

    h1 = _conv3x3(fsm, w1_ref, b1_ref[...], vm)
    h2 = _conv3x3(h1, w2_ref, b2_ref[...], vm)
    h3 = _conv3x3(h2, w3_ref, b3_ref[...], vm, relu=False)

    rows = []
    for i in range(NM):
        rows.append(h3 * (m[i:i + 1, :] * inv[i:i + 1, :]))
    B = jnp.concatenate(rows, axis=0)
    covs_ref[...] = _dot_t(B, h3)
    cnt_ref[...] = cnt


def _fc_body(sc_ref, cc_ref, sfw_ref, sfb_ref, cfw_ref, cfb_ref,
             sM_ref, cM_ref):
    sM_ref[...] = _dot_t(sc_ref[...], sfw_ref[...]) + sfb_ref[...]
    cM_ref[...] = _dot_t(cc_ref[...], cfw_ref[...]) + cfb_ref[...]


def _combine_body(sM_ref, cM_ref, cfsm_ref, sf_ref, cm_ref, sm_ref,
                  ccnt_ref, scnt_ref, compw_ref, compb_ref,
                  unzw_ref, unzb_ref, out_ref):
    cm = cm_ref[...]
    sm = sm_ref[...]
    ccnt = ccnt_ref[...]
    scnt = scnt_ref[...]

    ccf = _dot(compw_ref[...], cfsm_ref[...]) + compb_ref[...]

    sinv = 1.0 / jnp.maximum(scnt, 1.0)
    ssums = _dot_t(sf_ref[...], sm)

    valid = ((ccnt >= 10.0) & (scnt >= 10.0)).astype(jnp.float32)
    cond = cm * valid
    S = _last_valid_onehot(cond)
    anyS = jnp.sum(S, axis=0, keepdims=True)

    sMall = sM_ref[...]
    cMall = cM_ref[...]
    acc = ccf * (1.0 - anyS)
    for i in range(NM):
        tmat = _dot(sMall[i * 32:(i + 1) * 32, :],
                    cMall[i * 32:(i + 1) * 32, :])
        acc = acc + _dot(tmat, ccf) * S[i:i + 1, :]
    fsmean = _dot(ssums, S * sinv)
    out_ref[...] = _dot(unzw_ref[...], acc) + unzb_ref[...] + fsmean


def _branch(xf, m, w1, b1, w2, b2, w3, b3, vmask):
    covs, fsm, cnt = pl.pallas_call(
        _branch_body,
        out_shape=[
            jax.ShapeDtypeStruct((NM * 32, 32), jnp.float32),
            jax.ShapeDtypeStruct((256, HW), jnp.float32),
            jax.ShapeDtypeStruct((NM, 1), jnp.float32),
        ],
        interpret=_INTERPRET,
    )(xf, m, vmask, w1, b1, w2, b2, w3, b3)
    return covs, fsm, cnt


def kernel(cF, sF, cmasks, smasks, s_c1w, s_c1b, s_c2w, s_c2b, s_c3w, s_c3b,
           s_fcw, s_fcb, c_c1w, c_c1b, c_c2w, c_c2b, c_c3w, c_c3b, c_fcw,
           c_fcb, comp_w, comp_b, unzip_w, unzip_b):
    f32 = jnp.float32
    cmf = (cmasks[:, 0].reshape(NM, HW) == 1).astype(f32)
    smf = (smasks[:, 0].reshape(NM, HW) == 1).astype(f32)
    vmask = jnp.asarray(_VMASK_NP)

    def taps(w):
        return jnp.transpose(w, (2, 3, 0, 1)).reshape(9, w.shape[0], w.shape[1])

    cFf = cF.reshape(256, HW)
    sFf = sF.reshape(256, HW)

    scovs, _, scnt = _branch(sFf, smf, taps(s_c1w), s_c1b[:, None],
                             taps(s_c2w), s_c2b[:, None],
                             taps(s_c3w), s_c3b[:, None], vmask)
    ccovs, cfsm, ccnt = _branch(cFf, cmf, taps(c_c1w), c_c1b[:, None],
                                taps(c_c2w), c_c2b[:, None],
                                taps(c_c3w), c_c3b[:, None], vmask)

    sM, cM = pl.pallas_call(
        _fc_body,
        out_shape=[jax.ShapeDtypeStruct((NM, 1024), f32),
                   jax.ShapeDtypeStruct((NM, 1024), f32)],
        interpret=_INTERPRET,
    )(scovs.reshape(NM, 1024), ccovs.reshape(NM, 1024),
      s_fcw, s_fcb[None, :], c_fcw, c_fcb[None, :])

    out = pl.pallas_call(
        _combine_body,
        out_shape=jax.ShapeDtypeStruct((256, HW), f32),
        interpret=_INTERPRET,
    )(sM.reshape(NM * 32, 32), cM.reshape(NM * 32, 32), cfsm, sFf,
      cmf, smf, ccnt, scnt,
      comp_w.reshape(32, 256), comp_b[:, None],
      unzip_w.reshape(256, 32), unzip_b[:, None])

    return out.reshape(1, 256, H, W)
